# baseline (speedup 1.0000x reference)
import functools

import jax
import jax.numpy as jnp
from jax import lax
from jax.experimental import pallas as pl
from jax.experimental.pallas import tpu as pltpu
from jax.experimental.pallas import tpu_sc as plsc

N = 10000
E = 320000
SEQ = 8
L = 2
DIN = 128
DOUT = 64

NUM_CORES = 2
NUM_SUBCORES = 16
NUM_TILES = NUM_CORES * NUM_SUBCORES

CHE = 128
NCHUNK = E // CHE
N_ACC = 10048
RPT = N_ACC // NUM_TILES


@functools.lru_cache(maxsize=None)
def _make_seg_sum(d):
    mesh = plsc.VectorSubcoreMesh(core_axis_name="c", subcore_axis_name="s")

    @functools.partial(
        pl.kernel,
        out_type=jax.ShapeDtypeStruct((N_ACC, d), jnp.float32),
        mesh=mesh,
        compiler_params=pltpu.CompilerParams(use_tc_tiling_on_sc=False),
        scratch_types=[
            pltpu.VMEM((2, CHE), jnp.int32),
            pltpu.VMEM((2, CHE), jnp.int32),
            pltpu.VMEM((CHE, d), jnp.float32),
            pltpu.VMEM((CHE, d), jnp.float32),
            pltpu.VMEM((RPT, d), jnp.float32),
            pltpu.VMEM((16,), jnp.int32),
            pltpu.SemaphoreType.DMA,
            pltpu.SemaphoreType.DMA,
        ],
    )
    def seg_sum(feat_hbm, idx2_hbm, bounds_hbm, out_hbm, pair0_v, pair1_v,
                rows0_v, rows1_v, acc_v, bvec_v, sem0, sem1):
        cid = lax.axis_index("c")
        sid = lax.axis_index("s")
        wid = cid * NUM_SUBCORES + sid
        base = wid * RPT
        pairs = (pair0_v, pair1_v)
        rows = (rows0_v, rows1_v)
        sems = (sem0, sem1)

        pltpu.sync_copy(bounds_hbm.at[pl.ds(wid * 16, 16)], bvec_v)
        bv = bvec_v[...]
        lo = bv[0]
        hi = bv[1]
        cnt = hi - lo

        zvec = jnp.zeros((16,), jnp.float32)

        def zero_row(r, _):
            for j in range(d // 16):
                acc_v[r, pl.ds(j * 16, 16)] = zvec
            return 0

        lax.fori_loop(0, RPT, zero_row, 0)

        def accumulate(pair_v, rows_v, masked):
            def group(g, _):
                dvec = pair_v[1, pl.ds(g * 16, 16)]
                for l in range(16):
                    r = dvec[l] - base
                    e = g * 16 + l

                    def add():
                        for j in range(d // 16):
                            acc_v[r, pl.ds(j * 16, 16)] = (
                                acc_v[r, pl.ds(j * 16, 16)]
                                + rows_v[e, pl.ds(j * 16, 16)])

                    if masked:
                        pl.when(jnp.logical_and(r >= 0, r < RPT))(add)
                    else:
                        add()
                return 0

            lax.fori_loop(0, CHE // 16, group, 0)

        def do_chunk_sync(i, slot, masked):
            pltpu.sync_copy(idx2_hbm.at[i], pairs[slot])
            pltpu.async_copy(feat_hbm.at[pairs[slot].at[0]], rows[slot],
                             sems[slot]).wait()
            accumulate(pairs[slot], rows[slot], masked)

        pl.when(cnt >= 1)(lambda: do_chunk_sync(lo, 0, True))
        pl.when(cnt >= 2)(lambda: do_chunk_sync(hi - 1, 0, True))

        icnt = jnp.maximum(cnt - 2, 0)
        start = lo + 1

        def prime(b):
            pltpu.sync_copy(idx2_hbm.at[start + b], pairs[b])
            pltpu.async_copy(feat_hbm.at[pairs[b].at[0]], rows[b], sems[b])

        pl.when(icnt >= 1)(lambda: prime(0))
        pl.when(icnt >= 2)(lambda: prime(1))

        def outer(g, _):
            for b in range(2):
                jj = 2 * g + b

                def work():
                    pltpu.make_async_copy(
                        feat_hbm.at[pairs[b].at[0]], rows[b], sems[b]).wait()
                    accumulate(pairs[b], rows[b], False)

                    def refill():
                        pltpu.sync_copy(idx2_hbm.at[start + jj + 2], pairs[b])
                        pltpu.async_copy(feat_hbm.at[pairs[b].at[0]],
                                         rows[b], sems[b])

                    pl.when(jj + 2 < icnt)(refill)

                pl.when(jj < icnt)(work)
            return 0

        lax.fori_loop(0, (icnt + 1) // 2, outer, 0)

        pltpu.sync_copy(acc_v, out_hbm.at[pl.ds(base, RPT)])

    return seg_sum


def _cell_body(xin_ref, ax_ref, h_ref, ah_ref, deg_ref, wxs_ref, wxn_ref,
               whs_ref, whn_ref, b_ref, out_ref):
    inv = 1.0 / jnp.maximum(deg_ref[:, 0:1], 1.0)
    mx = ax_ref[...] * inv
    mh = ah_ref[...] * inv
    h = h_ref[...]
    pre = (jnp.dot(xin_ref[...], wxs_ref[...], preferred_element_type=jnp.float32)
           + jnp.dot(mx, wxn_ref[...], preferred_element_type=jnp.float32)
           + jnp.dot(h, whs_ref[...], preferred_element_type=jnp.float32)
           + jnp.dot(mh, whn_ref[...], preferred_element_type=jnp.float32)
           + b_ref[...])
    u = jax.nn.sigmoid(pre[:, :DOUT])
    c = jnp.tanh(pre[:, DOUT:])
    out_ref[...] = u * h + (1.0 - u) * c


@functools.lru_cache(maxsize=None)
def _make_cell(din):
    BLK = 1000
    grid = (N // BLK,)
    w2 = 2 * DOUT

    def rows(i):
        return (i, 0)

    def full2(i):
        return (0, 0)

    return pl.pallas_call(
        _cell_body,
        grid=grid,
        in_specs=[
            pl.BlockSpec((BLK, din), rows),
            pl.BlockSpec((BLK, din), rows),
            pl.BlockSpec((BLK, DOUT), rows),
            pl.BlockSpec((BLK, DOUT), rows),
            pl.BlockSpec((BLK, 16), rows),
            pl.BlockSpec((din, w2), full2),
            pl.BlockSpec((din, w2), full2),
            pl.BlockSpec((DOUT, w2), full2),
            pl.BlockSpec((DOUT, w2), full2),
            pl.BlockSpec((1, w2), full2),
        ],
        out_specs=pl.BlockSpec((BLK, DOUT), rows),
        out_shape=jax.ShapeDtypeStruct((N, DOUT), jnp.float32),
    )


def _edge_plan(edge_index):
    src = edge_index[0]
    dst = edge_index[1]
    order = jnp.argsort(dst)
    sdst = dst[order]
    ssrc = src[order]
    idx2 = jnp.stack([ssrc.reshape(NCHUNK, CHE), sdst.reshape(NCHUNK, CHE)],
                     axis=1)
    fences = jnp.searchsorted(
        sdst, jnp.arange(NUM_TILES + 1, dtype=jnp.int32) * RPT,
        side="left").astype(jnp.int32)
    lo = fences[:NUM_TILES] // CHE
    hi = -(-fences[1:] // CHE)
    bounds = (jnp.zeros((NUM_TILES, 16), jnp.int32)
              .at[:, 0].set(lo).at[:, 1].set(hi).reshape(-1))
    return idx2, bounds


def kernel(x, edge_index, hidden_states, Wx0_self, Wx0_neigh, bx0,
           Wx1_self, Wx1_neigh, bx1, Wh_self, Wh_neigh, bh):
    idx2, bounds = _edge_plan(edge_index)

    def cat(w):
        return jnp.concatenate([w[1], w[2]], axis=-1)

    wx_s = [cat(Wx0_self), cat(Wx1_self)]
    wx_n = [cat(Wx0_neigh), cat(Wx1_neigh)]
    wh_s = [cat(Wh_self[l]) for l in range(L)]
    wh_n = [cat(Wh_neigh[l]) for l in range(L)]
    bias = [(cat(bx0[:, None, :])[0] + cat(bh[0][:, None, :])[0])[None, :],
            (cat(bx1[:, None, :])[0] + cat(bh[1][:, None, :])[0])[None, :]]

    seg64 = _make_seg_sum(DOUT)
    seg128 = _make_seg_sum(DIN)
    seg16 = _make_seg_sum(16)
    cell0 = _make_cell(DIN)
    cell1 = _make_cell(DOUT)

    ones = jnp.ones((N, 16), jnp.float32)
    deg = seg16(ones, idx2, bounds)

    aggx = [seg128(x[i], idx2, bounds) for i in range(SEQ)]
    h0 = hidden_states[0]
    h1 = hidden_states[1]
    aggh0 = seg64(h0, idx2, bounds)
    aggh1 = seg64(h1, idx2, bounds)

    for i in range(SEQ):
        out0 = cell0(x[i], aggx[i], h0, aggh0, deg,
                     wx_s[0], wx_n[0], wh_s[0], wh_n[0], bias[0])
        agg_out0 = seg64(out0, idx2, bounds)
        out1 = cell1(out0, agg_out0, h1, aggh1, deg,
                     wx_s[1], wx_n[1], wh_s[1], wh_n[1], bias[1])
        h0, aggh0 = out0, agg_out0
        h1 = out1
        if i < SEQ - 1:
            aggh1 = seg64(out1, idx2, bounds)

    return (x, jnp.stack([h0, h1], axis=0))


# speedup vs baseline: 1.7087x; 1.7087x over previous
import functools

import jax
import jax.numpy as jnp
from jax import lax
from jax.experimental import pallas as pl
from jax.experimental.pallas import tpu as pltpu
from jax.experimental.pallas import tpu_sc as plsc

N = 10000
E = 320000
SEQ = 8
L = 2
DIN = 128
DOUT = 64

NUM_CORES = 2
NUM_SUBCORES = 16
NUM_TILES = NUM_CORES * NUM_SUBCORES

CHE = 128
NCHUNK = E // CHE
N_ACC = 10048
RPT = N_ACC // NUM_TILES


@functools.lru_cache(maxsize=None)
def _make_seg_sum(d, deg_mode=False):
    mesh = plsc.VectorSubcoreMesh(core_axis_name="c", subcore_axis_name="s")
    nj = d // 16

    @functools.partial(
        pl.kernel,
        out_type=jax.ShapeDtypeStruct((N_ACC, d), jnp.float32),
        mesh=mesh,
        compiler_params=pltpu.CompilerParams(use_tc_tiling_on_sc=False),
        scratch_types=[
            pltpu.VMEM((4, CHE), jnp.int32),
            pltpu.VMEM((4, CHE), jnp.int32),
            pltpu.VMEM((CHE, d), jnp.float32),
            pltpu.VMEM((CHE, d), jnp.float32),
            pltpu.VMEM((RPT, d), jnp.float32),
            pltpu.VMEM((16,), jnp.int32),
            pltpu.SemaphoreType.DMA,
            pltpu.SemaphoreType.DMA,
        ],
    )
    def seg_sum(feat_hbm, rec_hbm, bounds_hbm, out_hbm, rec0_v, rec1_v,
                rows0_v, rows1_v, acc_v, bvec_v, sem0, sem1):
        cid = lax.axis_index("c")
        sid = lax.axis_index("s")
        wid = cid * NUM_SUBCORES + sid
        base = wid * RPT
        recs = (rec0_v, rec1_v)
        rows = (rows0_v, rows1_v)
        sems = (sem0, sem1)

        pltpu.sync_copy(bounds_hbm.at[pl.ds(wid * 16, 16)], bvec_v)
        bv = bvec_v[...]
        lo = bv[0]
        hi = bv[1]
        cnt = hi - lo

        zvec = jnp.zeros((16,), jnp.float32)

        def zero_row(r, _):
            for j in range(nj):
                acc_v[r, pl.ds(j * 16, 16)] = zvec
            return 0

        lax.fori_loop(0, RPT, zero_row, 0)

        def process(rec_v, rows_v):
            ng = rec_v[3, pl.ds(0, 16)][0]

            def rgroup(gi, ptr):
                rr = rec_v[1, pl.ds(gi * 16, 16)]
                rc = rec_v[2, pl.ds(gi * 16, 16)]
                for l in range(16):
                    loc = rr[l] - base
                    rcnt = rc[l]

                    def do_run():
                        if deg_mode:
                            vals = (jnp.full((16,), 1.0, jnp.float32)
                                    * rcnt.astype(jnp.float32),)
                        else:
                            def edge(k, accs):
                                return tuple(
                                    accs[j] + rows_v[ptr + k,
                                                     pl.ds(j * 16, 16)]
                                    for j in range(nj))

                            vals = lax.fori_loop(
                                0, rcnt, edge,
                                tuple(zvec for _ in range(nj)))
                        for j in range(nj):
                            acc_v[loc, pl.ds(j * 16, 16)] = (
                                acc_v[loc, pl.ds(j * 16, 16)] + vals[j])

                    pl.when(jnp.logical_and(loc >= 0, loc < RPT))(do_run)
                    ptr = ptr + rcnt
                return ptr

            lax.fori_loop(0, ng, rgroup, 0)

        def load_rec(i, b):
            pltpu.sync_copy(rec_hbm.at[i], recs[b])
            if not deg_mode:
                pltpu.async_copy(feat_hbm.at[recs[b].at[0]], rows[b],
                                 sems[b])

        def prime(b):
            load_rec(lo + b, b)

        pl.when(cnt >= 1)(lambda: prime(0))
        pl.when(cnt >= 2)(lambda: prime(1))

        def outer(g, _):
            for b in range(2):
                jj = 2 * g + b

                def work():
                    if not deg_mode:
                        pltpu.make_async_copy(
                            feat_hbm.at[recs[b].at[0]], rows[b],
                            sems[b]).wait()
                    process(recs[b], rows[b])
                    pl.when(jj + 2 < cnt)(lambda: load_rec(lo + jj + 2, b))

                pl.when(jj < cnt)(work)
            return 0

        lax.fori_loop(0, (cnt + 1) // 2, outer, 0)

        pltpu.sync_copy(acc_v, out_hbm.at[pl.ds(base, RPT)])

    return seg_sum


def _cell_body(xin_ref, ax_ref, h_ref, ah_ref, deg_ref, wxs_ref, wxn_ref,
               whs_ref, whn_ref, b_ref, out_ref):
    inv = 1.0 / jnp.maximum(deg_ref[:, 0:1], 1.0)
    mx = ax_ref[...] * inv
    mh = ah_ref[...] * inv
    h = h_ref[...]
    pre = (jnp.dot(xin_ref[...], wxs_ref[...], preferred_element_type=jnp.float32)
           + jnp.dot(mx, wxn_ref[...], preferred_element_type=jnp.float32)
           + jnp.dot(h, whs_ref[...], preferred_element_type=jnp.float32)
           + jnp.dot(mh, whn_ref[...], preferred_element_type=jnp.float32)
           + b_ref[...])
    u = jax.nn.sigmoid(pre[:, :DOUT])
    c = jnp.tanh(pre[:, DOUT:])
    out_ref[...] = u * h + (1.0 - u) * c


@functools.lru_cache(maxsize=None)
def _make_cell(din):
    BLK = 1000
    grid = (N // BLK,)
    w2 = 2 * DOUT

    def rows(i):
        return (i, 0)

    def full2(i):
        return (0, 0)

    return pl.pallas_call(
        _cell_body,
        grid=grid,
        in_specs=[
            pl.BlockSpec((BLK, din), rows),
            pl.BlockSpec((BLK, din), rows),
            pl.BlockSpec((BLK, DOUT), rows),
            pl.BlockSpec((BLK, DOUT), rows),
            pl.BlockSpec((BLK, 16), rows),
            pl.BlockSpec((din, w2), full2),
            pl.BlockSpec((din, w2), full2),
            pl.BlockSpec((DOUT, w2), full2),
            pl.BlockSpec((DOUT, w2), full2),
            pl.BlockSpec((1, w2), full2),
        ],
        out_specs=pl.BlockSpec((BLK, DOUT), rows),
        out_shape=jax.ShapeDtypeStruct((N, DOUT), jnp.float32),
    )


def _edge_plan(edge_index):
    src = edge_index[0]
    dst = edge_index[1]
    order = jnp.argsort(dst)
    sdst = dst[order]
    ssrc = src[order]
    sd2 = sdst.reshape(NCHUNK, CHE)
    col = jnp.arange(CHE, dtype=jnp.int32)[None, :]
    first = jnp.concatenate(
        [jnp.ones((NCHUNK, 1), bool), sd2[:, 1:] != sd2[:, :-1]], axis=1)
    pos = jnp.where(first, col, 2 * CHE)
    sp = jnp.sort(pos, axis=1)
    spc = jnp.minimum(sp, CHE)
    sp_next = jnp.concatenate(
        [spc[:, 1:], jnp.full((NCHUNK, 1), CHE, jnp.int32)], axis=1)
    cnts = sp_next - spc
    valid = sp < CHE
    rrows = jnp.where(
        valid,
        jnp.take_along_axis(sd2, jnp.minimum(sp, CHE - 1), axis=1), -1)
    nruns = first.sum(axis=1, dtype=jnp.int32)
    ngroups = -(-nruns // 16)
    meta = jnp.zeros((NCHUNK, CHE), jnp.int32).at[:, 0].set(ngroups)
    rec = jnp.stack([ssrc.reshape(NCHUNK, CHE), rrows, cnts, meta], axis=1)
    fences = jnp.searchsorted(
        sdst, jnp.arange(NUM_TILES + 1, dtype=jnp.int32) * RPT,
        side="left").astype(jnp.int32)
    lo = fences[:NUM_TILES] // CHE
    hi = -(-fences[1:] // CHE)
    bounds = (jnp.zeros((NUM_TILES, 16), jnp.int32)
              .at[:, 0].set(lo).at[:, 1].set(hi).reshape(-1))
    return rec, bounds


def kernel(x, edge_index, hidden_states, Wx0_self, Wx0_neigh, bx0,
           Wx1_self, Wx1_neigh, bx1, Wh_self, Wh_neigh, bh):
    rec, bounds = _edge_plan(edge_index)

    def cat(w):
        return jnp.concatenate([w[1], w[2]], axis=-1)

    wx_s = [cat(Wx0_self), cat(Wx1_self)]
    wx_n = [cat(Wx0_neigh), cat(Wx1_neigh)]
    wh_s = [cat(Wh_self[l]) for l in range(L)]
    wh_n = [cat(Wh_neigh[l]) for l in range(L)]
    bias = [(cat(bx0[:, None, :])[0] + cat(bh[0][:, None, :])[0])[None, :],
            (cat(bx1[:, None, :])[0] + cat(bh[1][:, None, :])[0])[None, :]]

    seg64 = _make_seg_sum(DOUT)
    seg128 = _make_seg_sum(DIN)
    seg16 = _make_seg_sum(16, True)
    cell0 = _make_cell(DIN)
    cell1 = _make_cell(DOUT)

    ones = jnp.ones((N, 16), jnp.float32)
    deg = seg16(ones, rec, bounds)

    aggx = [seg128(x[i], rec, bounds) for i in range(SEQ)]
    h0 = hidden_states[0]
    h1 = hidden_states[1]
    aggh0 = seg64(h0, rec, bounds)
    aggh1 = seg64(h1, rec, bounds)

    for i in range(SEQ):
        out0 = cell0(x[i], aggx[i], h0, aggh0, deg,
                     wx_s[0], wx_n[0], wh_s[0], wh_n[0], bias[0])
        agg_out0 = seg64(out0, rec, bounds)
        out1 = cell1(out0, agg_out0, h1, aggh1, deg,
                     wx_s[1], wx_n[1], wh_s[1], wh_n[1], bias[1])
        h0, aggh0 = out0, agg_out0
        h1 = out1
        if i < SEQ - 1:
            aggh1 = seg64(out1, rec, bounds)

    return (x, jnp.stack([h0, h1], axis=0))
